# baseline (speedup 1.0000x reference)
import functools

import jax
import jax.numpy as jnp
from jax import lax
from jax.experimental import pallas as pl
from jax.experimental.pallas import tpu as pltpu
from jax.experimental.pallas import tpu_sc as plsc

F32 = jnp.float32
INV_SQRT_2 = 0.7071067811865476

N_ATOMS = 10000
N_EDGES = 320000
N_TRIP = 500000

N_AT_PAD = 10240
EDGE_BLK = 2000
TRIP_BLK = 2000
ATOM_BLK = 1280

NW = 32
SEG_CHUNK = 80
SEG_SLOTS = 4
SEG_NCHUNK = (N_EDGES // NW) // SEG_CHUNK
SEG_GROUPS = SEG_NCHUNK // SEG_SLOTS
TRIP_CHUNK = 248
TRIP_SLOTS = 4
TRIP_PAD = 507904
TRIP_NCHUNK = (TRIP_PAD // NW) // TRIP_CHUNK
TRIP_GROUPS = TRIP_NCHUNK // TRIP_SLOTS


def _silu(x):
    return x / (1.0 + jnp.exp(-x))


def _res(x, W1, W2):
    h = _silu(jnp.dot(x, W1, preferred_element_type=F32))
    h = _silu(jnp.dot(h, W2, preferred_element_type=F32))
    return (x + h) * INV_SQRT_2


def _edge_tbl_body(m_ref, rbf_ref, W_rbf_ref, Wab_ref, Wcd_ref, x_ref,
                   PQ_ref):
    m = m_ref[...]
    rbf = rbf_ref[...]
    x_ref[...] = m * jnp.dot(rbf, W_rbf_ref[...], preferred_element_type=F32)
    PQ_ref[...] = (jnp.dot(m, Wab_ref[...], preferred_element_type=F32)
                   + jnp.dot(rbf, Wcd_ref[...], preferred_element_type=F32))


def _edge_tbl_stage(m, rbf, W_rbf, Wab, Wcd):
    grid = N_EDGES // EDGE_BLK
    full = lambda w: pl.BlockSpec(w.shape, lambda i: (0,) * w.ndim)
    return pl.pallas_call(
        _edge_tbl_body,
        grid=(grid,),
        in_specs=[
            pl.BlockSpec((EDGE_BLK, 128), lambda i: (i, 0)),
            pl.BlockSpec((EDGE_BLK, 16), lambda i: (i, 0)),
            full(W_rbf), full(Wab), full(Wcd),
        ],
        out_specs=[
            pl.BlockSpec((EDGE_BLK, 128), lambda i: (i, 0)),
            pl.BlockSpec((EDGE_BLK, 128), lambda i: (i, 0)),
        ],
        out_shape=[
            jax.ShapeDtypeStruct((N_EDGES, 128), F32),
            jax.ShapeDtypeStruct((N_EDGES, 128), F32),
        ],
    )(m, rbf, W_rbf, Wab, Wcd)


def _force_body(m_ref, rbf_ref, W_f0_ref, Wf01_ref, Wf02_ref, Wf11_ref,
                Wf12_ref, W_rbf_F_ref, W_out_F_ref, F_ref):
    m = m_ref[...]
    rbf = rbf_ref[...]
    xF = _silu(jnp.dot(m, W_f0_ref[...], preferred_element_type=F32))
    xF = _res(xF, Wf01_ref[...], Wf02_ref[...])
    xF = _res(xF, Wf11_ref[...], Wf12_ref[...])
    xF = xF * jnp.dot(rbf, W_rbf_F_ref[...], preferred_element_type=F32)
    F_ref[...] = jnp.dot(xF, W_out_F_ref[...], preferred_element_type=F32)


def _force_stage(m, rbf, W_f0, Wf01, Wf02, Wf11, Wf12, W_rbf_F, W_out_F):
    grid = N_EDGES // EDGE_BLK
    full = lambda w: pl.BlockSpec(w.shape, lambda i: (0,) * w.ndim)
    return pl.pallas_call(
        _force_body,
        grid=(grid,),
        in_specs=[
            pl.BlockSpec((EDGE_BLK, 128), lambda i: (i, 0)),
            pl.BlockSpec((EDGE_BLK, 16), lambda i: (i, 0)),
            full(W_f0), full(Wf01), full(Wf02), full(Wf11), full(Wf12),
            full(W_rbf_F), full(W_out_F),
        ],
        out_specs=[pl.BlockSpec((EDGE_BLK, 1), lambda i: (i, 0))],
        out_shape=[jax.ShapeDtypeStruct((N_EDGES, 1), F32)],
    )(m, rbf, W_f0, Wf01, Wf02, Wf11, Wf12, W_rbf_F, W_out_F)[0]


def _seg_sum_body(x_hbm, idj_hbm, out_hbm, *sc):
    idx = sc[0:SEG_SLOTS]
    rows = sc[SEG_SLOTS:2 * SEG_SLOTS]
    acc_sh = sc[2 * SEG_SLOTS]
    sem = sc[2 * SEG_SLOTS + 1:2 * SEG_SLOTS + 1 + SEG_SLOTS]
    c = lax.axis_index("c")
    s = lax.axis_index("s")
    wid = s * 2 + c
    def zrow(i, _):
        def zcol(j, _):
            rows[0][i, pl.ds(j * 16, 16)] = jnp.zeros((16,), F32)
            return 0
        return lax.fori_loop(0, 8, zcol, 0)
    lax.fori_loop(0, SEG_CHUNK, zrow, 0)
    def zacc(k, _):
        pltpu.sync_copy(rows[0],
                        acc_sh.at[pl.ds(s * 640 + k * SEG_CHUNK, SEG_CHUNK)])
        return 0
    lax.fori_loop(0, 640 // SEG_CHUNK, zacc, 0)
    plsc.subcore_barrier()
    base = wid * (N_EDGES // NW)
    ns = SEG_SLOTS
    def group(g, _):
        offs = [base + (g * ns + b) * SEG_CHUNK for b in range(ns)]
        di = [pltpu.async_copy(idj_hbm.at[pl.ds(offs[b], SEG_CHUNK)], idx[b],
                               sem[b]) for b in range(ns)]
        dr = [pltpu.async_copy(x_hbm.at[pl.ds(offs[b], SEG_CHUNK)], rows[b],
                               sem[b]) for b in range(ns)]
        st = []
        for b in range(ns):
            di[b].wait()
            dr[b].wait()
            st.append(pltpu.async_copy(rows[b], acc_sh.at[idx[b]], sem[b],
                                       add=True))
        for b in range(ns):
            st[b].wait()
        return 0
    lax.fori_loop(0, SEG_GROUPS, group, 0)
    def rem(k, _):
        off = base + k * SEG_CHUNK
        pltpu.sync_copy(idj_hbm.at[pl.ds(off, SEG_CHUNK)], idx[0])
        pltpu.sync_copy(x_hbm.at[pl.ds(off, SEG_CHUNK)], rows[0])
        pltpu.sync_copy(rows[0], acc_sh.at[idx[0]], add=True)
        return 0
    lax.fori_loop(SEG_GROUPS * ns, SEG_NCHUNK, rem, 0)
    plsc.subcore_barrier()
    pltpu.sync_copy(acc_sh.at[pl.ds(s * 640, 640)],
                    out_hbm.at[c].at[pl.ds(s * 640, 640)])


def _trip_gather_body(tbl_hbm, i_hbm, j_hbm, out_hbm, *sc):
    ii = sc[0:TRIP_SLOTS]
    jj = sc[TRIP_SLOTS:2 * TRIP_SLOTS]
    buf = sc[2 * TRIP_SLOTS:3 * TRIP_SLOTS]
    sem = sc[3 * TRIP_SLOTS:4 * TRIP_SLOTS]
    c = lax.axis_index("c")
    s = lax.axis_index("s")
    wid = s * 2 + c
    base = wid * TRIP_NCHUNK * TRIP_CHUNK
    ns = TRIP_SLOTS
    def group(g, _):
        offs = [base + (g * ns + b) * TRIP_CHUNK for b in range(ns)]
        di = [pltpu.async_copy(i_hbm.at[pl.ds(offs[b], TRIP_CHUNK)], ii[b],
                               sem[b]) for b in range(ns)]
        dj = [pltpu.async_copy(j_hbm.at[pl.ds(offs[b], TRIP_CHUNK)], jj[b],
                               sem[b]) for b in range(ns)]
        gp = []
        for b in range(ns):
            di[b].wait()
            dj[b].wait()
            gp.append(pltpu.async_copy(tbl_hbm.at[ii[b]], buf[b], sem[b]))
        gq = []
        for b in range(ns):
            gp[b].wait()
            gq.append(pltpu.async_copy(tbl_hbm.at[jj[b]], buf[b], sem[b],
                                       add=True))
        st = []
        for b in range(ns):
            gq[b].wait()
            st.append(pltpu.async_copy(buf[b],
                                       out_hbm.at[pl.ds(offs[b], TRIP_CHUNK)],
                                       sem[b]))
        for b in range(ns):
            st[b].wait()
        return 0
    lax.fori_loop(0, TRIP_GROUPS, group, 0)


@functools.lru_cache(maxsize=1)
def _sc_kernels():
    mesh = plsc.VectorSubcoreMesh(core_axis_name="c", subcore_axis_name="s")
    seg = pl.kernel(
        _seg_sum_body,
        out_type=jax.ShapeDtypeStruct((2, N_AT_PAD, 128), F32),
        mesh=mesh,
        scratch_types=(
            [pltpu.VMEM((SEG_CHUNK,), jnp.int32)] * SEG_SLOTS
            + [pltpu.VMEM((SEG_CHUNK, 128), F32)] * SEG_SLOTS
            + [pltpu.VMEM_SHARED((N_AT_PAD, 128), F32)]
            + [pltpu.SemaphoreType.DMA] * SEG_SLOTS
        ),
    )
    trip = pl.kernel(
        _trip_gather_body,
        out_type=jax.ShapeDtypeStruct((TRIP_PAD, 64), F32),
        mesh=mesh,
        compiler_params=pltpu.CompilerParams(use_tc_tiling_on_sc=False),
        scratch_types=(
            [pltpu.VMEM((TRIP_CHUNK,), jnp.int32)] * TRIP_SLOTS
            + [pltpu.VMEM((TRIP_CHUNK,), jnp.int32)] * TRIP_SLOTS
            + [pltpu.VMEM((TRIP_CHUNK, 64), F32)] * TRIP_SLOTS
            + [pltpu.SemaphoreType.DMA] * TRIP_SLOTS
        ),
    )
    return seg, trip


def _trip_body(pre_ref, cbf_ref, Wce_ref, Ws1_ref, Ws2_ref, S_ref):
    t = pre_ref[...] + jnp.dot(cbf_ref[...], Wce_ref[...],
                               preferred_element_type=F32)
    xs = _silu(t)
    xs = _silu(jnp.dot(xs, Ws1_ref[...], preferred_element_type=F32))
    S_ref[...] = jnp.dot(xs, Ws2_ref[...], preferred_element_type=F32)


def _trip_stage(pre, cbf, Wce, Ws1, Ws2):
    grid = N_TRIP // TRIP_BLK
    full = lambda w: pl.BlockSpec(w.shape, lambda i: (0,) * w.ndim)
    return pl.pallas_call(
        _trip_body,
        grid=(grid,),
        in_specs=[
            pl.BlockSpec((TRIP_BLK, 64), lambda i: (i, 0)),
            pl.BlockSpec((TRIP_BLK, 16), lambda i: (i, 0)),
            full(Wce), full(Ws1), full(Ws2),
        ],
        out_specs=[pl.BlockSpec((TRIP_BLK, 6), lambda i: (i, 0))],
        out_shape=[jax.ShapeDtypeStruct((N_TRIP, 6), F32)],
    )(pre, cbf, Wce, Ws1, Ws2)[0]


def _atom_body(a_ref, b_ref, W_e0_ref, We01_ref, We02_ref, We11_ref,
               We12_ref, W_out_E_ref, E_ref):
    xE = a_ref[...] + b_ref[...]
    xE = _silu(jnp.dot(xE, W_e0_ref[...], preferred_element_type=F32))
    xE = _res(xE, We01_ref[...], We02_ref[...])
    xE = _res(xE, We11_ref[...], We12_ref[...])
    E_ref[...] = jnp.dot(xE, W_out_E_ref[...], preferred_element_type=F32)


def _atom_stage(xa, xb, W_e0, We01, We02, We11, We12, W_out_E):
    grid = N_AT_PAD // ATOM_BLK
    full = lambda w: pl.BlockSpec(w.shape, lambda i: (0,) * w.ndim)
    return pl.pallas_call(
        _atom_body,
        grid=(grid,),
        in_specs=[
            pl.BlockSpec((ATOM_BLK, 128), lambda i: (i, 0)),
            pl.BlockSpec((ATOM_BLK, 128), lambda i: (i, 0)),
            full(W_e0), full(We01), full(We02), full(We11), full(We12),
            full(W_out_E),
        ],
        out_specs=[pl.BlockSpec((ATOM_BLK, 1), lambda i: (i, 0))],
        out_shape=[jax.ShapeDtypeStruct((N_AT_PAD, 1), F32)],
    )(xa, xb, W_e0, We01, We02, We11, We12, W_out_E)[0]


def kernel(h, m, rbf, cbf, id_j, id3_i, id3_j,
           W_rbf, W_e0, We_r0_1, We_r0_2, We_r1_1, We_r1_2, W_out_E,
           W_f0, Wf_r0_1, Wf_r0_2, Wf_r1_1, Wf_r1_2, W_rbf_F, W_out_F,
           Ws0, Ws1, Ws2):
    del h
    Wab = jnp.concatenate([Ws0[0:128], Ws0[128:256]], axis=1)
    Wcd = jnp.concatenate([Ws0[256:272], Ws0[272:288]], axis=1)
    Wce = Ws0[288:304]

    x, PQ = _edge_tbl_stage(m, rbf, W_rbf, Wab, Wcd)

    seg_sum, trip_gather = _sc_kernels()

    pad = TRIP_PAD - N_TRIP
    i32 = jnp.concatenate([id3_i.astype(jnp.int32) * 2,
                           jnp.zeros((pad,), jnp.int32)])
    j32 = jnp.concatenate([id3_j.astype(jnp.int32) * 2 + 1,
                           jnp.zeros((pad,), jnp.int32)])
    pre = trip_gather(PQ.reshape(2 * N_EDGES, 64), i32, j32)

    idj32 = id_j.astype(jnp.int32)
    xE2 = seg_sum(x, idj32)

    F = _force_stage(m, rbf, W_f0, Wf_r0_1, Wf_r0_2, Wf_r1_1, Wf_r1_2,
                     W_rbf_F, W_out_F)
    S = _trip_stage(pre, cbf, Wce, Ws1, Ws2)
    E_full = _atom_stage(xE2[0], xE2[1], W_e0, We_r0_1, We_r0_2,
                         We_r1_1, We_r1_2, W_out_E)
    return (E_full[:N_ATOMS], F, S)


# speedup vs baseline: 1.1169x; 1.1169x over previous
import functools

import jax
import jax.numpy as jnp
from jax import lax
from jax.experimental import pallas as pl
from jax.experimental.pallas import tpu as pltpu
from jax.experimental.pallas import tpu_sc as plsc

F32 = jnp.float32
INV_SQRT_2 = 0.7071067811865476

N_ATOMS = 10000
N_EDGES = 320000
N_TRIP = 500000

N_AT_PAD = 10240
EDGE_BLK = 2000
TRIP_BLK = 2000
ATOM_BLK = 1280

NW = 32
SEG_CHUNK = 80
SEG_SLOTS = 4
SEG_NCHUNK = (N_EDGES // NW) // SEG_CHUNK
SEG_GROUPS = SEG_NCHUNK // SEG_SLOTS
TRIP_CHUNK = 248
TRIP_SLOTS = 4
TRIP_PAD = 507904
TRIP_NCHUNK = (TRIP_PAD // NW) // TRIP_CHUNK
TRIP_GROUPS = TRIP_NCHUNK // TRIP_SLOTS


def _silu(x):
    return x / (1.0 + jnp.exp(-x))


def _res(x, W1, W2):
    h = _silu(jnp.dot(x, W1, preferred_element_type=F32))
    h = _silu(jnp.dot(h, W2, preferred_element_type=F32))
    return (x + h) * INV_SQRT_2


def _edge_tbl_body(m_ref, rbf_ref, W_rbf_ref, Wab_ref, Wcd_ref, x_ref,
                   PQ_ref):
    m = m_ref[...]
    rbf = rbf_ref[...]
    x_ref[...] = m * jnp.dot(rbf, W_rbf_ref[...], preferred_element_type=F32)
    PQ_ref[...] = (jnp.dot(m, Wab_ref[...], preferred_element_type=F32)
                   + jnp.dot(rbf, Wcd_ref[...], preferred_element_type=F32))


def _edge_tbl_stage(m, rbf, W_rbf, Wab, Wcd):
    grid = N_EDGES // EDGE_BLK
    full = lambda w: pl.BlockSpec(w.shape, lambda i: (0,) * w.ndim)
    return pl.pallas_call(
        _edge_tbl_body,
        grid=(grid,),
        in_specs=[
            pl.BlockSpec((EDGE_BLK, 128), lambda i: (i, 0)),
            pl.BlockSpec((EDGE_BLK, 16), lambda i: (i, 0)),
            full(W_rbf), full(Wab), full(Wcd),
        ],
        out_specs=[
            pl.BlockSpec((EDGE_BLK, 128), lambda i: (i, 0)),
            pl.BlockSpec((EDGE_BLK, 128), lambda i: (i, 0)),
        ],
        out_shape=[
            jax.ShapeDtypeStruct((N_EDGES, 128), F32),
            jax.ShapeDtypeStruct((N_EDGES, 128), F32),
        ],
    )(m, rbf, W_rbf, Wab, Wcd)


def _force_body(m_ref, rbf_ref, W_f0_ref, Wf01_ref, Wf02_ref, Wf11_ref,
                Wf12_ref, W_rbf_F_ref, W_out_F_ref, F_ref):
    m = m_ref[...]
    rbf = rbf_ref[...]
    xF = _silu(jnp.dot(m, W_f0_ref[...], preferred_element_type=F32))
    xF = _res(xF, Wf01_ref[...], Wf02_ref[...])
    xF = _res(xF, Wf11_ref[...], Wf12_ref[...])
    xF = xF * jnp.dot(rbf, W_rbf_F_ref[...], preferred_element_type=F32)
    F_ref[...] = jnp.dot(xF, W_out_F_ref[...], preferred_element_type=F32)


def _force_stage(m, rbf, W_f0, Wf01, Wf02, Wf11, Wf12, W_rbf_F, W_out_F):
    grid = N_EDGES // EDGE_BLK
    full = lambda w: pl.BlockSpec(w.shape, lambda i: (0,) * w.ndim)
    return pl.pallas_call(
        _force_body,
        grid=(grid,),
        in_specs=[
            pl.BlockSpec((EDGE_BLK, 128), lambda i: (i, 0)),
            pl.BlockSpec((EDGE_BLK, 16), lambda i: (i, 0)),
            full(W_f0), full(Wf01), full(Wf02), full(Wf11), full(Wf12),
            full(W_rbf_F), full(W_out_F),
        ],
        out_specs=[pl.BlockSpec((EDGE_BLK, 1), lambda i: (i, 0))],
        out_shape=[jax.ShapeDtypeStruct((N_EDGES, 1), F32)],
    )(m, rbf, W_f0, Wf01, Wf02, Wf11, Wf12, W_rbf_F, W_out_F)[0]


def _seg_sum_body(x_hbm, idj_hbm, out_hbm, *sc):
    idx = sc[0:SEG_SLOTS]
    rows = sc[SEG_SLOTS:2 * SEG_SLOTS]
    acc_sh = sc[2 * SEG_SLOTS]
    sem = sc[2 * SEG_SLOTS + 1:2 * SEG_SLOTS + 1 + SEG_SLOTS]
    c = lax.axis_index("c")
    s = lax.axis_index("s")
    wid = s * 2 + c
    def zrow(i, _):
        def zcol(j, _):
            rows[0][i, pl.ds(j * 16, 16)] = jnp.zeros((16,), F32)
            return 0
        return lax.fori_loop(0, 8, zcol, 0)
    lax.fori_loop(0, SEG_CHUNK, zrow, 0)
    def zacc(k, _):
        pltpu.sync_copy(rows[0],
                        acc_sh.at[pl.ds(s * 640 + k * SEG_CHUNK, SEG_CHUNK)])
        return 0
    lax.fori_loop(0, 640 // SEG_CHUNK, zacc, 0)
    plsc.subcore_barrier()
    base = wid * (N_EDGES // NW)
    ns = SEG_SLOTS
    def group(g, _):
        offs = [base + (g * ns + b) * SEG_CHUNK for b in range(ns)]
        di = [pltpu.async_copy(idj_hbm.at[pl.ds(offs[b], SEG_CHUNK)], idx[b],
                               sem[b]) for b in range(ns)]
        dr = [pltpu.async_copy(x_hbm.at[pl.ds(offs[b], SEG_CHUNK)], rows[b],
                               sem[b]) for b in range(ns)]
        st = []
        for b in range(ns):
            di[b].wait()
            dr[b].wait()
            st.append(pltpu.async_copy(rows[b], acc_sh.at[idx[b]], sem[b],
                                       add=True))
        for b in range(ns):
            st[b].wait()
        return 0
    lax.fori_loop(0, SEG_GROUPS, group, 0)
    def rem(k, _):
        off = base + k * SEG_CHUNK
        pltpu.sync_copy(idj_hbm.at[pl.ds(off, SEG_CHUNK)], idx[0])
        pltpu.sync_copy(x_hbm.at[pl.ds(off, SEG_CHUNK)], rows[0])
        pltpu.sync_copy(rows[0], acc_sh.at[idx[0]], add=True)
        return 0
    lax.fori_loop(SEG_GROUPS * ns, SEG_NCHUNK, rem, 0)
    plsc.subcore_barrier()
    pltpu.sync_copy(acc_sh.at[pl.ds(s * 640, 640)],
                    out_hbm.at[c].at[pl.ds(s * 640, 640)])


def _trip_gather_body(tbl_hbm, i_hbm, j_hbm, out_hbm, *sc):
    ii = sc[0:TRIP_SLOTS]
    jj = sc[TRIP_SLOTS:2 * TRIP_SLOTS]
    buf = sc[2 * TRIP_SLOTS:3 * TRIP_SLOTS]
    sem = sc[3 * TRIP_SLOTS:4 * TRIP_SLOTS]
    c = lax.axis_index("c")
    s = lax.axis_index("s")
    wid = s * 2 + c
    base = wid * TRIP_NCHUNK * TRIP_CHUNK
    ns = TRIP_SLOTS
    def group(g, _):
        offs = [base + (g * ns + b) * TRIP_CHUNK for b in range(ns)]
        di = [pltpu.async_copy(i_hbm.at[pl.ds(offs[b], TRIP_CHUNK)], ii[b],
                               sem[b]) for b in range(ns)]
        dj = [pltpu.async_copy(j_hbm.at[pl.ds(offs[b], TRIP_CHUNK)], jj[b],
                               sem[b]) for b in range(ns)]
        gp = []
        for b in range(ns):
            di[b].wait()
            dj[b].wait()
            gp.append(pltpu.async_copy(tbl_hbm.at[ii[b]], buf[b], sem[b]))
        gq = []
        for b in range(ns):
            gp[b].wait()
            gq.append(pltpu.async_copy(tbl_hbm.at[jj[b]], buf[b], sem[b],
                                       add=True))
        st = []
        for b in range(ns):
            gq[b].wait()
            st.append(pltpu.async_copy(buf[b],
                                       out_hbm.at[pl.ds(offs[b], TRIP_CHUNK)],
                                       sem[b]))
        for b in range(ns):
            st[b].wait()
        return 0
    lax.fori_loop(0, TRIP_GROUPS, group, 0)


@functools.lru_cache(maxsize=1)
def _sc_kernels():
    mesh = plsc.VectorSubcoreMesh(core_axis_name="c", subcore_axis_name="s")
    seg = pl.kernel(
        _seg_sum_body,
        out_type=jax.ShapeDtypeStruct((2, N_AT_PAD, 128), F32),
        mesh=mesh,
        scratch_types=(
            [pltpu.VMEM((SEG_CHUNK,), jnp.int32)] * SEG_SLOTS
            + [pltpu.VMEM((SEG_CHUNK, 128), F32)] * SEG_SLOTS
            + [pltpu.VMEM_SHARED((N_AT_PAD, 128), F32)]
            + [pltpu.SemaphoreType.DMA] * SEG_SLOTS
        ),
    )
    trip = pl.kernel(
        _trip_gather_body,
        out_type=jax.ShapeDtypeStruct((TRIP_PAD, 64), F32),
        mesh=mesh,
        compiler_params=pltpu.CompilerParams(use_tc_tiling_on_sc=False),
        scratch_types=(
            [pltpu.VMEM((TRIP_CHUNK,), jnp.int32)] * TRIP_SLOTS
            + [pltpu.VMEM((TRIP_CHUNK,), jnp.int32)] * TRIP_SLOTS
            + [pltpu.VMEM((TRIP_CHUNK, 64), F32)] * TRIP_SLOTS
            + [pltpu.SemaphoreType.DMA] * TRIP_SLOTS
        ),
    )
    return seg, trip


def _trip_body(pre_ref, cbf_ref, Wce2_ref, Ws1b_ref, Ws2b_ref, S_ref):
    t = pre_ref[...] + jnp.dot(cbf_ref[...], Wce2_ref[...],
                               preferred_element_type=F32)
    xs = _silu(t)
    xs = _silu(jnp.dot(xs, Ws1b_ref[...], preferred_element_type=F32))
    S_ref[...] = jnp.dot(xs, Ws2b_ref[...], preferred_element_type=F32)


def _block_diag2(W):
    z = jnp.zeros_like(W)
    return jnp.concatenate(
        [jnp.concatenate([W, z], axis=1), jnp.concatenate([z, W], axis=1)],
        axis=0)


def _trip_stage(pre2, cbf2, Wce2, Ws1b, Ws2b):
    half = N_TRIP // 2
    grid = half // TRIP_BLK
    full = lambda w: pl.BlockSpec(w.shape, lambda i: (0,) * w.ndim)
    return pl.pallas_call(
        _trip_body,
        grid=(grid,),
        in_specs=[
            pl.BlockSpec((TRIP_BLK, 128), lambda i: (i, 0)),
            pl.BlockSpec((TRIP_BLK, 32), lambda i: (i, 0)),
            full(Wce2), full(Ws1b), full(Ws2b),
        ],
        out_specs=[pl.BlockSpec((TRIP_BLK, 12), lambda i: (i, 0))],
        out_shape=[jax.ShapeDtypeStruct((half, 12), F32)],
    )(pre2, cbf2, Wce2, Ws1b, Ws2b)[0]


def _atom_body(a_ref, b_ref, W_e0_ref, We01_ref, We02_ref, We11_ref,
               We12_ref, W_out_E_ref, E_ref):
    xE = a_ref[...] + b_ref[...]
    xE = _silu(jnp.dot(xE, W_e0_ref[...], preferred_element_type=F32))
    xE = _res(xE, We01_ref[...], We02_ref[...])
    xE = _res(xE, We11_ref[...], We12_ref[...])
    E_ref[...] = jnp.dot(xE, W_out_E_ref[...], preferred_element_type=F32)


def _atom_stage(xa, xb, W_e0, We01, We02, We11, We12, W_out_E):
    grid = N_AT_PAD // ATOM_BLK
    full = lambda w: pl.BlockSpec(w.shape, lambda i: (0,) * w.ndim)
    return pl.pallas_call(
        _atom_body,
        grid=(grid,),
        in_specs=[
            pl.BlockSpec((ATOM_BLK, 128), lambda i: (i, 0)),
            pl.BlockSpec((ATOM_BLK, 128), lambda i: (i, 0)),
            full(W_e0), full(We01), full(We02), full(We11), full(We12),
            full(W_out_E),
        ],
        out_specs=[pl.BlockSpec((ATOM_BLK, 1), lambda i: (i, 0))],
        out_shape=[jax.ShapeDtypeStruct((N_AT_PAD, 1), F32)],
    )(xa, xb, W_e0, We01, We02, We11, We12, W_out_E)[0]


def kernel(h, m, rbf, cbf, id_j, id3_i, id3_j,
           W_rbf, W_e0, We_r0_1, We_r0_2, We_r1_1, We_r1_2, W_out_E,
           W_f0, Wf_r0_1, Wf_r0_2, Wf_r1_1, Wf_r1_2, W_rbf_F, W_out_F,
           Ws0, Ws1, Ws2):
    del h
    Wab = jnp.concatenate([Ws0[0:128], Ws0[128:256]], axis=1)
    Wcd = jnp.concatenate([Ws0[256:272], Ws0[272:288]], axis=1)
    Wce = Ws0[288:304]

    x, PQ = _edge_tbl_stage(m, rbf, W_rbf, Wab, Wcd)

    seg_sum, trip_gather = _sc_kernels()

    pad = TRIP_PAD - N_TRIP
    i32 = jnp.concatenate([id3_i.astype(jnp.int32) * 2,
                           jnp.zeros((pad,), jnp.int32)])
    j32 = jnp.concatenate([id3_j.astype(jnp.int32) * 2 + 1,
                           jnp.zeros((pad,), jnp.int32)])
    pre = trip_gather(PQ.reshape(2 * N_EDGES, 64), i32, j32)

    idj32 = id_j.astype(jnp.int32)
    xE2 = seg_sum(x, idj32)

    F = _force_stage(m, rbf, W_f0, Wf_r0_1, Wf_r0_2, Wf_r1_1, Wf_r1_2,
                     W_rbf_F, W_out_F)
    pre2 = pre.reshape(TRIP_PAD // 2, 128)
    cbf2 = cbf.reshape(N_TRIP // 2, 2 * 16)
    S2 = _trip_stage(pre2, cbf2, _block_diag2(Wce), _block_diag2(Ws1),
                     _block_diag2(Ws2))
    S = S2.reshape(N_TRIP, 6)
    E_full = _atom_stage(xE2[0], xE2[1], W_e0, We_r0_1, We_r0_2,
                         We_r1_1, We_r1_2, W_out_E)
    return (E_full[:N_ATOMS], F, S)
